# Initial kernel scaffold; baseline (speedup 1.0000x reference)
import jax
import jax.numpy as jnp
from jax.experimental import pallas as pl


def kernel(x, edge_index, edge_attr, y, W1, att1, W2, att2, fc1_w, fc1_b, fc2_w, fc2_b, fc3_w, fc3_b, fc4_w, fc4_b):
    raise NotImplementedError("write your pallas kernel here")


# speedup vs baseline: 3.2626x; 3.2626x over previous
import functools

import jax
import jax.numpy as jnp
from jax import lax
from jax.experimental import pallas as pl
from jax.experimental.pallas import tpu as pltpu
from jax.experimental.pallas import tpu_sc as plsc

N = 10000
E = 320000
F_IN = 128
P = 16
H1 = 16
H2 = 4

NC = 2
NS = 16
L = 16
C = 400
EPW = E // (NC * NS)
EPT = E // NS
NCH1 = EPW // C
NCH2 = EPT // C
NPAD = 10240
RPS = NPAD // NS
C1 = 80
NCHA1 = EPT // C1


def _leaky(v):
    return jnp.where(v > 0, v, v * jnp.float32(0.2))


def _elu(v):
    return jnp.where(v > 0, v, jnp.exp(v) - jnp.float32(1.0))


def _sc_segsum_body(dst_h, src_h, ea_h, sd_h, ss_h, out_h,
                    s_sh, sd_t, ss_t, dst_v, src_v, ea_v, score_v, ex_v):
    cid = lax.axis_index("c")
    sid = lax.axis_index("s")
    z = jnp.zeros((L,), jnp.float32)

    def zb(i, _):
        ex_v[i, :] = z
        return 0
    lax.fori_loop(0, C, zb, 0)
    r0 = sid * RPS
    pltpu.sync_copy(ex_v, s_sh.at[pl.ds(r0, C)])
    pltpu.sync_copy(ex_v.at[pl.ds(0, RPS - C)], s_sh.at[pl.ds(r0 + C, RPS - C)])
    nn = sd_h.shape[0]
    pltpu.sync_copy(sd_h, sd_t.at[pl.ds(0, nn)])
    pltpu.sync_copy(ss_h, ss_t.at[pl.ds(0, nn)])
    plsc.subcore_barrier()

    base = (cid * NS + sid) * EPW

    def chunk(k, _):
        off = base + k * C
        pltpu.sync_copy(dst_h.at[pl.ds(off, C)], dst_v)
        pltpu.sync_copy(src_h.at[pl.ds(off, C)], src_v)
        pltpu.sync_copy(ea_h.at[pl.ds(off, C)], ea_v)
        for g in range(C // L):
            d16 = dst_v[pl.ds(g * L, L)]
            s16 = src_v[pl.ds(g * L, L)]
            sc = plsc.load_gather(sd_t, [d16]) + plsc.load_gather(ss_t, [s16])
            score_v[pl.ds(g * L, L)] = _leaky(sc)

        def edge(e, _):
            e16 = jnp.full((L,), e, jnp.int32)
            sv = plsc.load_gather(score_v, [e16])
            ex_v[e, :] = jnp.exp(ea_v[e, :] * sv)
            return 0
        lax.fori_loop(0, C, edge, 0)
        pltpu.sync_copy(ex_v, s_sh.at[dst_v], add=True)
        return 0
    lax.fori_loop(0, NCH1, chunk, 0)
    plsc.subcore_barrier()
    pltpu.sync_copy(s_sh.at[pl.ds(r0, RPS)], out_h.at[cid, pl.ds(r0, RPS)])


def _make_sc_segsum(mesh):
    return pl.kernel(
        _sc_segsum_body,
        out_type=jax.ShapeDtypeStruct((NC, NPAD, P), jnp.float32),
        mesh=mesh,
        compiler_params=pltpu.CompilerParams(needs_layout_passes=False, use_tc_tiling_on_sc=False),
        scratch_types=[
            pltpu.VMEM_SHARED((NPAD, P), jnp.float32),
            pltpu.VMEM((NPAD,), jnp.float32),
            pltpu.VMEM((NPAD,), jnp.float32),
            pltpu.VMEM((C,), jnp.int32),
            pltpu.VMEM((C,), jnp.int32),
            pltpu.VMEM((C, P), jnp.float32),
            pltpu.VMEM((C,), jnp.float32),
            pltpu.VMEM((C, P), jnp.float32),
        ],
    )


def _sc_agg1_body(dst_h, src_h, ea_h, sd_h, ss_h, s_h, h_h, alpha_h, out_h,
                  x_sh, sd_t, ss_t, dst_v, src_v, ea_v, score_v,
                  srow_v, hrow_v, alpha_v, msg_v, sem1, sem2):
    cid = lax.axis_index("c")
    sid = lax.axis_index("s")
    coff = cid * (P // NC)
    z = jnp.zeros((L,), jnp.float32)

    def zb(i, _):
        for j in range(8):
            msg_v[i, pl.ds(j * L, L)] = z
        return 0
    lax.fori_loop(0, C1, zb, 0)
    r0 = sid * RPS
    for t in range(RPS // C1):
        pltpu.sync_copy(msg_v, x_sh.at[pl.ds(r0 + t * C1, C1)])
    nn = sd_h.shape[0]
    pltpu.sync_copy(sd_h, sd_t.at[pl.ds(0, nn)])
    pltpu.sync_copy(ss_h, ss_t.at[pl.ds(0, nn)])
    plsc.subcore_barrier()

    base = sid * EPT

    def chunk(k, _):
        off = base + k * C1
        pltpu.sync_copy(dst_h.at[pl.ds(off, C1)], dst_v)
        pltpu.sync_copy(src_h.at[pl.ds(off, C1)], src_v)
        pltpu.sync_copy(ea_h.at[pl.ds(off, C1)], ea_v)
        for g in range(C1 // L):
            d16 = dst_v[pl.ds(g * L, L)]
            s16 = src_v[pl.ds(g * L, L)]
            sc = plsc.load_gather(sd_t, [d16]) + plsc.load_gather(ss_t, [s16])
            score_v[pl.ds(g * L, L)] = _leaky(sc)
        pltpu.sync_copy(s_h.at[dst_v], srow_v)
        pltpu.sync_copy(h_h.at[src_v], hrow_v)

        def edge(e, _):
            e16 = jnp.full((L,), e, jnp.int32)
            sv = plsc.load_gather(score_v, [e16])
            ex = jnp.exp(ea_v[e, :] * sv)
            al = ex / (srow_v[e, :] + jnp.float32(1e-16))
            alpha_v[e, :] = al
            hr = hrow_v[e, :]
            for j in range(8):
                j16 = jnp.full((L,), coff + j, jnp.int32)
                a = plsc.load_gather(alpha_v, [e16, j16])
                msg_v[e, pl.ds(j * L, L)] = hr * a
            return 0
        lax.fori_loop(0, C1, edge, 0)
        pltpu.sync_copy(msg_v, x_sh.at[dst_v], add=True)

        @pl.when(cid == 0)
        def _():
            pltpu.sync_copy(alpha_v, alpha_h.at[pl.ds(off, C1)])
        return 0
    lax.fori_loop(0, NCHA1, chunk, 0)
    plsc.subcore_barrier()
    pltpu.sync_copy(x_sh.at[pl.ds(r0, RPS)], out_h.at[cid, pl.ds(r0, RPS)])


def _make_sc_agg1(mesh):
    return pl.kernel(
        _sc_agg1_body,
        out_type=(jax.ShapeDtypeStruct((E, P), jnp.float32),
                  jax.ShapeDtypeStruct((NC, NPAD, (P // NC) * H1), jnp.float32)),
        mesh=mesh,
        compiler_params=pltpu.CompilerParams(needs_layout_passes=False, use_tc_tiling_on_sc=False),
        scratch_types=[
        pltpu.VMEM_SHARED((NPAD, (P // NC) * H1), jnp.float32),
        pltpu.VMEM((NPAD,), jnp.float32),
        pltpu.VMEM((NPAD,), jnp.float32),
        pltpu.VMEM((C1,), jnp.int32),
        pltpu.VMEM((C1,), jnp.int32),
        pltpu.VMEM((C1, P), jnp.float32),
        pltpu.VMEM((C1,), jnp.float32),
        pltpu.VMEM((C1, P), jnp.float32),
        pltpu.VMEM((C1, H1), jnp.float32),
        pltpu.VMEM((C1, P), jnp.float32),
        pltpu.VMEM((C1, (P // NC) * H1), jnp.float32),
        pltpu.SemaphoreType.DMA,
        pltpu.SemaphoreType.DMA,
    ],
)


def _sc_agg2_body(dst_h, src_h, ea_h, sd_h, ss_h, s_h, h_h, out_h,
                  x_sh, sd_t, ss_t, dst_v, src_v, ea_v, score_v,
                  srow_v, hrow_v, alpha_v, msg_v, sem1, sem2):
    cid = lax.axis_index("c")
    sid = lax.axis_index("s")
    coff = cid * (P // NC)
    z = jnp.zeros((L,), jnp.float32)
    iota = lax.iota(jnp.int32, L)
    rep4 = iota // 4
    tile4 = iota % 4

    def zb(i, _):
        msg_v[i, pl.ds(0, L)] = z
        msg_v[i, pl.ds(L, L)] = z
        return 0
    lax.fori_loop(0, C, zb, 0)
    r0 = sid * RPS
    pltpu.sync_copy(msg_v, x_sh.at[pl.ds(r0, C)])
    pltpu.sync_copy(msg_v.at[pl.ds(0, RPS - C)], x_sh.at[pl.ds(r0 + C, RPS - C)])
    nn = sd_h.shape[0]
    pltpu.sync_copy(sd_h, sd_t.at[pl.ds(0, nn)])
    pltpu.sync_copy(ss_h, ss_t.at[pl.ds(0, nn)])
    plsc.subcore_barrier()

    base = sid * EPT

    def chunk(k, _):
        off = base + k * C
        pltpu.sync_copy(dst_h.at[pl.ds(off, C)], dst_v)
        pltpu.sync_copy(src_h.at[pl.ds(off, C)], src_v)
        pltpu.sync_copy(ea_h.at[pl.ds(off, C)], ea_v)
        for g in range(C // L):
            d16 = dst_v[pl.ds(g * L, L)]
            s16 = src_v[pl.ds(g * L, L)]
            sc = plsc.load_gather(sd_t, [d16]) + plsc.load_gather(ss_t, [s16])
            score_v[pl.ds(g * L, L)] = _leaky(sc)
        pltpu.sync_copy(s_h.at[dst_v], srow_v)
        pltpu.sync_copy(h_h.at[src_v], hrow_v)

        def edge(e, _):
            e16 = jnp.full((L,), e, jnp.int32)
            sv = plsc.load_gather(score_v, [e16])
            ex = jnp.exp(ea_v[e, :] * sv)
            al = ex / (srow_v[e, :] + jnp.float32(1e-16))
            alpha_v[e, :] = al
            hv = plsc.load_gather(hrow_v, [e16, tile4])
            a0 = plsc.load_gather(alpha_v, [e16, coff + rep4])
            a1 = plsc.load_gather(alpha_v, [e16, coff + 4 + rep4])
            msg_v[e, pl.ds(0, L)] = a0 * hv
            msg_v[e, pl.ds(L, L)] = a1 * hv
            return 0
        lax.fori_loop(0, C, edge, 0)
        pltpu.sync_copy(msg_v, x_sh.at[dst_v], add=True)
        return 0
    lax.fori_loop(0, NCH2, chunk, 0)
    plsc.subcore_barrier()
    pltpu.sync_copy(x_sh.at[pl.ds(r0, RPS)], out_h.at[cid, pl.ds(r0, RPS)])


def _make_sc_agg2(mesh):
    return pl.kernel(
        _sc_agg2_body,
        out_type=jax.ShapeDtypeStruct((NC, NPAD, (P // NC) * H2), jnp.float32),
        mesh=mesh,
        compiler_params=pltpu.CompilerParams(needs_layout_passes=False, use_tc_tiling_on_sc=False),
        scratch_types=[
        pltpu.VMEM_SHARED((NPAD, (P // NC) * H2), jnp.float32),
        pltpu.VMEM((NPAD,), jnp.float32),
        pltpu.VMEM((NPAD,), jnp.float32),
        pltpu.VMEM((C,), jnp.int32),
        pltpu.VMEM((C,), jnp.int32),
        pltpu.VMEM((C, P), jnp.float32),
        pltpu.VMEM((C,), jnp.float32),
        pltpu.VMEM((C, P), jnp.float32),
        pltpu.VMEM((C, P), jnp.float32),
        pltpu.VMEM((C, P), jnp.float32),
        pltpu.VMEM((C, (P // NC) * H2), jnp.float32),
        pltpu.SemaphoreType.DMA,
        pltpu.SemaphoreType.DMA,
    ],
)


@functools.lru_cache(maxsize=None)
def _sc_kernels():
    mesh = plsc.VectorSubcoreMesh(core_axis_name="c", subcore_axis_name="s",
                                  num_cores=NC, num_subcores=NS)
    return (_make_sc_segsum(mesh), _make_sc_agg1(mesh), _make_sc_agg2(mesh))


NB = 10
NBK = N // NB
NBKP = NPAD // NB


def _tc1_body(x_ref, w_ref, adt_ref, h_ref, sd_ref, ss_ref):
    h = jnp.dot(x_ref[...], w_ref[...], preferred_element_type=jnp.float32)
    h_ref[...] = h
    sdss = lax.dot_general(h, adt_ref[...], (((1,), (1,)), ((), ())),
                           preferred_element_type=jnp.float32)
    sd_ref[...] = sdss[:, 0:1]
    ss_ref[...] = sdss[:, 1:2]


def _tc1(x, W1, adt):
    return pl.pallas_call(
        _tc1_body,
        grid=(NB,),
        in_specs=[
            pl.BlockSpec((NBK, F_IN), lambda i: (i, 0)),
            pl.BlockSpec((F_IN, H1), lambda i: (0, 0)),
            pl.BlockSpec((2, H1), lambda i: (0, 0)),
        ],
        out_specs=[
            pl.BlockSpec((NBK, H1), lambda i: (i, 0)),
            pl.BlockSpec((NBK, 1), lambda i: (i, 0)),
            pl.BlockSpec((NBK, 1), lambda i: (i, 0)),
        ],
        out_shape=[
            jax.ShapeDtypeStruct((N, H1), jnp.float32),
            jax.ShapeDtypeStruct((N, 1), jnp.float32),
            jax.ShapeDtypeStruct((N, 1), jnp.float32),
        ],
    )(x, W1, adt)


def _tc_sum_body(p_ref, o_ref):
    o_ref[...] = p_ref[0] + p_ref[1]


def _tc_sum(parts):
    return pl.pallas_call(
        _tc_sum_body,
        grid=(NB,),
        in_specs=[pl.BlockSpec((NC, NBKP, P), lambda i: (0, i, 0))],
        out_specs=pl.BlockSpec((NBKP, P), lambda i: (i, 0)),
        out_shape=jax.ShapeDtypeStruct((NPAD, P), jnp.float32),
    )(parts)


def _tc3_body(p_ref, w2_ref, adt2_ref, h2p_ref, sd_ref, ss_ref):
    a = _elu(p_ref[0])
    b = _elu(p_ref[1])
    h2 = (jnp.dot(a, w2_ref[0:128, :], preferred_element_type=jnp.float32)
          + jnp.dot(b, w2_ref[128:256, :], preferred_element_type=jnp.float32))
    h2p_ref[...] = jnp.concatenate(
        [h2, jnp.zeros((NBKP, P - H2), jnp.float32)], axis=1)
    sdss = lax.dot_general(h2, adt2_ref[...], (((1,), (1,)), ((), ())),
                           preferred_element_type=jnp.float32)
    sd_ref[...] = sdss[:, 0:1]
    ss_ref[...] = sdss[:, 1:2]


def _tc3(x1p, W2, adt2):
    return pl.pallas_call(
        _tc3_body,
        grid=(NB,),
        in_specs=[
            pl.BlockSpec((NC, NBKP, (P // NC) * H1), lambda i: (0, i, 0)),
            pl.BlockSpec((P * H1, H2), lambda i: (0, 0)),
            pl.BlockSpec((2, H2), lambda i: (0, 0)),
        ],
        out_specs=[
            pl.BlockSpec((NBKP, P), lambda i: (i, 0)),
            pl.BlockSpec((NBKP, 1), lambda i: (i, 0)),
            pl.BlockSpec((NBKP, 1), lambda i: (i, 0)),
        ],
        out_shape=[
            jax.ShapeDtypeStruct((NPAD, P), jnp.float32),
            jax.ShapeDtypeStruct((NPAD, 1), jnp.float32),
            jax.ShapeDtypeStruct((NPAD, 1), jnp.float32),
        ],
    )(x1p, W2, adt2)


NBH = 20
KB = (N * P * H2) // NBH


def _head_body(xf_ref, w1_ref, b1_ref, w2_ref, b2_ref, w3_ref, b3_ref,
               w4_ref, b4_ref, out_ref, acc_ref):
    i = pl.program_id(0)
    v = _elu(xf_ref[...])
    p = jnp.dot(v, w1_ref[...], preferred_element_type=jnp.float32)

    @pl.when(i == 0)
    def _():
        acc_ref[...] = p

    @pl.when(i > 0)
    def _():
        acc_ref[...] = acc_ref[...] + p

    @pl.when(i == NBH - 1)
    def _():
        zz = _elu(acc_ref[...] + b1_ref[...])
        zz = _elu(jnp.dot(zz, w2_ref[...], preferred_element_type=jnp.float32)
                  + b2_ref[...])
        zz = _elu(jnp.dot(zz, w3_ref[...], preferred_element_type=jnp.float32)
                  + b3_ref[...])
        out_ref[...] = (jnp.dot(zz, w4_ref[...],
                                preferred_element_type=jnp.float32)
                        + b4_ref[...])


def _tc_head(xf, w1, b1, w2, b2, w3, b3, w4, b4):
    return pl.pallas_call(
        _head_body,
        grid=(NBH,),
        in_specs=[
            pl.BlockSpec((1, KB), lambda i: (0, i)),
            pl.BlockSpec((KB, 20), lambda i: (i, 0)),
            pl.BlockSpec((1, 20), lambda i: (0, 0)),
            pl.BlockSpec((20, 16), lambda i: (0, 0)),
            pl.BlockSpec((1, 16), lambda i: (0, 0)),
            pl.BlockSpec((16, 20), lambda i: (0, 0)),
            pl.BlockSpec((1, 20), lambda i: (0, 0)),
            pl.BlockSpec((20, 2), lambda i: (0, 0)),
            pl.BlockSpec((1, 2), lambda i: (0, 0)),
        ],
        out_specs=pl.BlockSpec((1, 2), lambda i: (0, 0)),
        out_shape=jax.ShapeDtypeStruct((1, 2), jnp.float32),
        scratch_shapes=[pltpu.VMEM((1, 20), jnp.float32)],
    )(xf, w1, b1, w2, b2, w3, b3, w4, b4)


def kernel(x, edge_index, edge_attr, y, W1, att1, W2, att2,
           fc1_w, fc1_b, fc2_w, fc2_b, fc3_w, fc3_b, fc4_w, fc4_b):
    src = edge_index[0]
    dst = edge_index[1]
    _sc_segsum, _sc_agg1, _sc_agg2 = _sc_kernels()

    h1, sd1, ss1 = _tc1(x, W1, att1.reshape(2, H1))
    sd1 = sd1.reshape(N)
    ss1 = ss1.reshape(N)

    s1p = _sc_segsum(dst, src, edge_attr, sd1, ss1)
    s1 = _tc_sum(s1p)
    alpha1, x1p = _sc_agg1(dst, src, edge_attr, sd1, ss1, s1, h1)

    h2p, sd2, ss2 = _tc3(x1p, W2, att2.reshape(2, H2))
    sd2 = sd2.reshape(NPAD)
    ss2 = ss2.reshape(NPAD)

    s2p = _sc_segsum(dst, src, alpha1, sd2, ss2)
    s2 = _tc_sum(s2p)
    x2p = _sc_agg2(dst, src, alpha1, sd2, ss2, s2, h2p)

    xf = jnp.transpose(x2p, (1, 0, 2)).reshape(1, NPAD * P * H2)
    out = _tc_head(xf, fc1_w, fc1_b.reshape(1, 20),
                   fc2_w, fc2_b.reshape(1, 16), fc3_w, fc3_b.reshape(1, 20),
                   fc4_w, fc4_b.reshape(1, 2))
    reg = jnp.zeros((1,), jnp.float32)
    return (out, reg)


# speedup vs baseline: 3.6419x; 1.1163x over previous
import functools

import jax
import jax.numpy as jnp
from jax import lax
from jax.experimental import pallas as pl
from jax.experimental.pallas import tpu as pltpu
from jax.experimental.pallas import tpu_sc as plsc

N = 10000
E = 320000
F_IN = 128
P = 16
H1 = 16
H2 = 4

NC = 2
NS = 16
L = 16
EPT = E // NS
NPAD = 10240
RPS = NPAD // NS
C1 = 80
NCH1 = EPT // C1
C2 = 400
NCH2 = EPT // C2
EPS = 1e-16

_params = pltpu.CompilerParams(needs_layout_passes=False,
                               use_tc_tiling_on_sc=False)


def _leaky(v):
    return jnp.where(v > 0, v, v * jnp.float32(0.2))


def _elu(v):
    return jnp.where(v > 0, v, jnp.exp(v) - jnp.float32(1.0))


def _sc_l1_body(dst_h, src_h, ea_h, sd_h, ss_h, h_h, x_out, s_out,
                x_sh, s_sh, sd_t, ss_t, dst_v, src_v, ea_v, score_v,
                hrow_v, ex_v, msg_v):
    cid = lax.axis_index("c")
    sid = lax.axis_index("s")
    coff = cid * (P // NC)
    z = jnp.zeros((L,), jnp.float32)

    def zb(i, _):
        ex_v[i, :] = z
        for j in range(8):
            msg_v[i, pl.ds(j * L, L)] = z
        return 0
    lax.fori_loop(0, C1, zb, 0)
    r0 = sid * RPS
    for t in range(RPS // C1):
        pltpu.sync_copy(msg_v, x_sh.at[pl.ds(r0 + t * C1, C1)])
        pltpu.sync_copy(ex_v, s_sh.at[pl.ds(r0 + t * C1, C1)])
    nn = sd_h.shape[0]
    pltpu.sync_copy(sd_h, sd_t.at[pl.ds(0, nn)])
    pltpu.sync_copy(ss_h, ss_t.at[pl.ds(0, nn)])
    plsc.subcore_barrier()

    base = sid * EPT

    def chunk(k, _):
        off = base + k * C1
        pltpu.sync_copy(dst_h.at[pl.ds(off, C1)], dst_v)
        pltpu.sync_copy(src_h.at[pl.ds(off, C1)], src_v)
        pltpu.sync_copy(ea_h.at[pl.ds(off, C1)], ea_v)
        pltpu.sync_copy(h_h.at[src_v], hrow_v)
        for g in range(C1 // L):
            d16 = dst_v[pl.ds(g * L, L)]
            s16 = src_v[pl.ds(g * L, L)]
            sc = plsc.load_gather(sd_t, [d16]) + plsc.load_gather(ss_t, [s16])
            score_v[pl.ds(g * L, L)] = _leaky(sc)

        def edge(e, _):
            e16 = jnp.full((L,), e, jnp.int32)
            sv = plsc.load_gather(score_v, [e16])
            ex_v[e, :] = jnp.exp(ea_v[e, :] * sv)
            hr = hrow_v[e, :]
            for j in range(8):
                j16 = jnp.full((L,), coff + j, jnp.int32)
                a = plsc.load_gather(ex_v, [e16, j16])
                msg_v[e, pl.ds(j * L, L)] = hr * a
            return 0
        lax.fori_loop(0, C1, edge, 0)
        pltpu.sync_copy(ex_v, s_sh.at[dst_v], add=True)
        pltpu.sync_copy(msg_v, x_sh.at[dst_v], add=True)
        return 0
    lax.fori_loop(0, NCH1, chunk, 0)
    plsc.subcore_barrier()

    for t in range(RPS // C1):
        rr = r0 + t * C1
        pltpu.sync_copy(x_sh.at[pl.ds(rr, C1)], msg_v)
        pltpu.sync_copy(s_sh.at[pl.ds(rr, C1)], ex_v)

        def nrm(r, _):
            r16 = jnp.full((L,), r, jnp.int32)
            for j in range(8):
                j16 = jnp.full((L,), coff + j, jnp.int32)
                sv = plsc.load_gather(ex_v, [r16, j16]) + jnp.float32(EPS)
                msg_v[r, pl.ds(j * L, L)] = msg_v[r, pl.ds(j * L, L)] / sv
            return 0
        lax.fori_loop(0, C1, nrm, 0)
        pltpu.sync_copy(msg_v, x_out.at[cid, pl.ds(rr, C1)])

    @pl.when(cid == 0)
    def _():
        pltpu.sync_copy(s_sh.at[pl.ds(r0, RPS)], s_out.at[pl.ds(r0, RPS)])


def _make_sc_l1(mesh):
    return pl.kernel(
        _sc_l1_body,
        out_type=(jax.ShapeDtypeStruct((NC, NPAD, (P // NC) * H1), jnp.float32),
                  jax.ShapeDtypeStruct((NPAD, P), jnp.float32)),
        mesh=mesh,
        compiler_params=_params,
        scratch_types=[
            pltpu.VMEM_SHARED((NPAD, (P // NC) * H1), jnp.float32),
            pltpu.VMEM_SHARED((NPAD, P), jnp.float32),
            pltpu.VMEM((NPAD,), jnp.float32),
            pltpu.VMEM((NPAD,), jnp.float32),
            pltpu.VMEM((C1,), jnp.int32),
            pltpu.VMEM((C1,), jnp.int32),
            pltpu.VMEM((C1, P), jnp.float32),
            pltpu.VMEM((C1,), jnp.float32),
            pltpu.VMEM((C1, H1), jnp.float32),
            pltpu.VMEM((C1, P), jnp.float32),
            pltpu.VMEM((C1, (P // NC) * H1), jnp.float32),
        ],
    )


def _sc_l2_body(dst_h, src_h, ea_h, sd1_h, ss1_h, s1_h, sd2_h, ss2_h, h_h,
                x_out, x_sh, s_sh, sd1_t, ss1_t, sd2_t, ss2_t,
                dst_v, src_v, ea_v, sc1_v, sc2_v, s1row_v, hrow_v,
                ex_v, msg_v):
    cid = lax.axis_index("c")
    sid = lax.axis_index("s")
    coff = cid * (P // NC)
    z = jnp.zeros((L,), jnp.float32)
    iota = lax.iota(jnp.int32, L)
    rep4 = iota // 4
    tile4 = iota % 4

    def zb(i, _):
        ex_v[i, :] = z
        msg_v[i, pl.ds(0, L)] = z
        msg_v[i, pl.ds(L, L)] = z
        return 0
    lax.fori_loop(0, C2, zb, 0)
    r0 = sid * RPS
    pltpu.sync_copy(msg_v, x_sh.at[pl.ds(r0, C2)])
    pltpu.sync_copy(msg_v.at[pl.ds(0, RPS - C2)],
                    x_sh.at[pl.ds(r0 + C2, RPS - C2)])
    pltpu.sync_copy(ex_v, s_sh.at[pl.ds(r0, C2)])
    pltpu.sync_copy(ex_v.at[pl.ds(0, RPS - C2)],
                    s_sh.at[pl.ds(r0 + C2, RPS - C2)])
    n1 = sd1_h.shape[0]
    pltpu.sync_copy(sd1_h, sd1_t.at[pl.ds(0, n1)])
    pltpu.sync_copy(ss1_h, ss1_t.at[pl.ds(0, n1)])
    pltpu.sync_copy(sd2_h, sd2_t)
    pltpu.sync_copy(ss2_h, ss2_t)
    plsc.subcore_barrier()

    base = sid * EPT

    def chunk(k, _):
        off = base + k * C2
        pltpu.sync_copy(dst_h.at[pl.ds(off, C2)], dst_v)
        pltpu.sync_copy(src_h.at[pl.ds(off, C2)], src_v)
        pltpu.sync_copy(ea_h.at[pl.ds(off, C2)], ea_v)
        pltpu.sync_copy(s1_h.at[dst_v], s1row_v)
        pltpu.sync_copy(h_h.at[src_v], hrow_v)
        for g in range(C2 // L):
            d16 = dst_v[pl.ds(g * L, L)]
            s16 = src_v[pl.ds(g * L, L)]
            c1 = plsc.load_gather(sd1_t, [d16]) + plsc.load_gather(ss1_t, [s16])
            sc1_v[pl.ds(g * L, L)] = _leaky(c1)
            c2 = plsc.load_gather(sd2_t, [d16]) + plsc.load_gather(ss2_t, [s16])
            sc2_v[pl.ds(g * L, L)] = _leaky(c2)

        def edge(e, _):
            e16 = jnp.full((L,), e, jnp.int32)
            sv1 = plsc.load_gather(sc1_v, [e16])
            ex1 = jnp.exp(ea_v[e, :] * sv1)
            al1 = ex1 / (s1row_v[e, :] + jnp.float32(EPS))
            sv2 = plsc.load_gather(sc2_v, [e16])
            ex_v[e, :] = jnp.exp(al1 * sv2)
            hv = plsc.load_gather(hrow_v, [e16, tile4])
            a0 = plsc.load_gather(ex_v, [e16, coff + rep4])
            a1 = plsc.load_gather(ex_v, [e16, coff + 4 + rep4])
            msg_v[e, pl.ds(0, L)] = a0 * hv
            msg_v[e, pl.ds(L, L)] = a1 * hv
            return 0
        lax.fori_loop(0, C2, edge, 0)
        pltpu.sync_copy(ex_v, s_sh.at[dst_v], add=True)
        pltpu.sync_copy(msg_v, x_sh.at[dst_v], add=True)
        return 0
    lax.fori_loop(0, NCH2, chunk, 0)
    plsc.subcore_barrier()

    sizes = (C2, RPS - C2)
    offs = (0, C2)
    for t in range(2):
        rr = r0 + offs[t]
        sz = sizes[t]
        pltpu.sync_copy(x_sh.at[pl.ds(rr, sz)], msg_v.at[pl.ds(0, sz)])
        pltpu.sync_copy(s_sh.at[pl.ds(rr, sz)], ex_v.at[pl.ds(0, sz)])

        def nrm(r, _):
            r16 = jnp.full((L,), r, jnp.int32)
            for j in range(2):
                jrep = coff + j * 4 + rep4
                sv = plsc.load_gather(ex_v, [r16, jrep]) + jnp.float32(EPS)
                msg_v[r, pl.ds(j * L, L)] = msg_v[r, pl.ds(j * L, L)] / sv
            return 0
        lax.fori_loop(0, sz, nrm, 0)
        pltpu.sync_copy(msg_v.at[pl.ds(0, sz)], x_out.at[cid, pl.ds(rr, sz)])


def _make_sc_l2(mesh):
    return pl.kernel(
        _sc_l2_body,
        out_type=jax.ShapeDtypeStruct((NC, NPAD, (P // NC) * H2), jnp.float32),
        mesh=mesh,
        compiler_params=_params,
        scratch_types=[
            pltpu.VMEM_SHARED((NPAD, (P // NC) * H2), jnp.float32),
            pltpu.VMEM_SHARED((NPAD, P), jnp.float32),
            pltpu.VMEM((NPAD,), jnp.float32),
            pltpu.VMEM((NPAD,), jnp.float32),
            pltpu.VMEM((NPAD,), jnp.float32),
            pltpu.VMEM((NPAD,), jnp.float32),
            pltpu.VMEM((C2,), jnp.int32),
            pltpu.VMEM((C2,), jnp.int32),
            pltpu.VMEM((C2, P), jnp.float32),
            pltpu.VMEM((C2,), jnp.float32),
            pltpu.VMEM((C2,), jnp.float32),
            pltpu.VMEM((C2, P), jnp.float32),
            pltpu.VMEM((C2, P), jnp.float32),
            pltpu.VMEM((C2, P), jnp.float32),
            pltpu.VMEM((C2, (P // NC) * H2), jnp.float32),
        ],
    )


@functools.lru_cache(maxsize=None)
def _sc_kernels():
    mesh = plsc.VectorSubcoreMesh(core_axis_name="c", subcore_axis_name="s",
                                  num_cores=NC, num_subcores=NS)
    return (_make_sc_l1(mesh), _make_sc_l2(mesh))


NB = 10
NBK = N // NB
NBKP = NPAD // NB


def _tc1_body(x_ref, w_ref, adt_ref, h_ref, sd_ref, ss_ref):
    h = jnp.dot(x_ref[...], w_ref[...], preferred_element_type=jnp.float32)
    h_ref[...] = h
    sdss = lax.dot_general(h, adt_ref[...], (((1,), (1,)), ((), ())),
                           preferred_element_type=jnp.float32)
    sd_ref[...] = sdss[:, 0:1]
    ss_ref[...] = sdss[:, 1:2]


def _tc1(x, W1, adt):
    return pl.pallas_call(
        _tc1_body,
        grid=(NB,),
        in_specs=[
            pl.BlockSpec((NBK, F_IN), lambda i: (i, 0)),
            pl.BlockSpec((F_IN, H1), lambda i: (0, 0)),
            pl.BlockSpec((2, H1), lambda i: (0, 0)),
        ],
        out_specs=[
            pl.BlockSpec((NBK, H1), lambda i: (i, 0)),
            pl.BlockSpec((NBK, 1), lambda i: (i, 0)),
            pl.BlockSpec((NBK, 1), lambda i: (i, 0)),
        ],
        out_shape=[
            jax.ShapeDtypeStruct((N, H1), jnp.float32),
            jax.ShapeDtypeStruct((N, 1), jnp.float32),
            jax.ShapeDtypeStruct((N, 1), jnp.float32),
        ],
    )(x, W1, adt)


def _tc3_body(p_ref, w2_ref, adt2_ref, h2p_ref, sd_ref, ss_ref):
    a = _elu(p_ref[0])
    b = _elu(p_ref[1])
    h2 = (jnp.dot(a, w2_ref[0:128, :], preferred_element_type=jnp.float32)
          + jnp.dot(b, w2_ref[128:256, :], preferred_element_type=jnp.float32))
    h2p_ref[...] = jnp.concatenate(
        [h2, jnp.zeros((NBKP, P - H2), jnp.float32)], axis=1)
    sdss = lax.dot_general(h2, adt2_ref[...], (((1,), (1,)), ((), ())),
                           preferred_element_type=jnp.float32)
    sd_ref[...] = sdss[:, 0:1]
    ss_ref[...] = sdss[:, 1:2]


def _tc3(x1p, W2, adt2):
    return pl.pallas_call(
        _tc3_body,
        grid=(NB,),
        in_specs=[
            pl.BlockSpec((NC, NBKP, (P // NC) * H1), lambda i: (0, i, 0)),
            pl.BlockSpec((P * H1, H2), lambda i: (0, 0)),
            pl.BlockSpec((2, H2), lambda i: (0, 0)),
        ],
        out_specs=[
            pl.BlockSpec((NBKP, P), lambda i: (i, 0)),
            pl.BlockSpec((NBKP, 1), lambda i: (i, 0)),
            pl.BlockSpec((NBKP, 1), lambda i: (i, 0)),
        ],
        out_shape=[
            jax.ShapeDtypeStruct((NPAD, P), jnp.float32),
            jax.ShapeDtypeStruct((NPAD, 1), jnp.float32),
            jax.ShapeDtypeStruct((NPAD, 1), jnp.float32),
        ],
    )(x1p, W2, adt2)


NBH = 20
KB = (N * P * H2) // NBH


def _head_body(xf_ref, w1_ref, b1_ref, w2_ref, b2_ref, w3_ref, b3_ref,
               w4_ref, b4_ref, out_ref, acc_ref):
    i = pl.program_id(0)
    v = _elu(xf_ref[...])
    p = jnp.dot(v, w1_ref[...], preferred_element_type=jnp.float32)

    @pl.when(i == 0)
    def _():
        acc_ref[...] = p

    @pl.when(i > 0)
    def _():
        acc_ref[...] = acc_ref[...] + p

    @pl.when(i == NBH - 1)
    def _():
        zz = _elu(acc_ref[...] + b1_ref[...])
        zz = _elu(jnp.dot(zz, w2_ref[...], preferred_element_type=jnp.float32)
                  + b2_ref[...])
        zz = _elu(jnp.dot(zz, w3_ref[...], preferred_element_type=jnp.float32)
                  + b3_ref[...])
        out_ref[...] = (jnp.dot(zz, w4_ref[...],
                                preferred_element_type=jnp.float32)
                        + b4_ref[...])


def _tc_head(xf, w1, b1, w2, b2, w3, b3, w4, b4):
    return pl.pallas_call(
        _head_body,
        grid=(NBH,),
        in_specs=[
            pl.BlockSpec((1, KB), lambda i: (0, i)),
            pl.BlockSpec((KB, 20), lambda i: (i, 0)),
            pl.BlockSpec((1, 20), lambda i: (0, 0)),
            pl.BlockSpec((20, 16), lambda i: (0, 0)),
            pl.BlockSpec((1, 16), lambda i: (0, 0)),
            pl.BlockSpec((16, 20), lambda i: (0, 0)),
            pl.BlockSpec((1, 20), lambda i: (0, 0)),
            pl.BlockSpec((20, 2), lambda i: (0, 0)),
            pl.BlockSpec((1, 2), lambda i: (0, 0)),
        ],
        out_specs=pl.BlockSpec((1, 2), lambda i: (0, 0)),
        out_shape=jax.ShapeDtypeStruct((1, 2), jnp.float32),
        scratch_shapes=[pltpu.VMEM((1, 20), jnp.float32)],
    )(xf, w1, b1, w2, b2, w3, b3, w4, b4)


def kernel(x, edge_index, edge_attr, y, W1, att1, W2, att2,
           fc1_w, fc1_b, fc2_w, fc2_b, fc3_w, fc3_b, fc4_w, fc4_b):
    src = edge_index[0]
    dst = edge_index[1]
    _sc_l1, _sc_l2 = _sc_kernels()

    h1, sd1, ss1 = _tc1(x, W1, att1.reshape(2, H1))
    sd1 = sd1.reshape(N)
    ss1 = ss1.reshape(N)

    x1p, s1 = _sc_l1(dst, src, edge_attr, sd1, ss1, h1)

    h2p, sd2, ss2 = _tc3(x1p, W2, att2.reshape(2, H2))
    sd2 = sd2.reshape(NPAD)
    ss2 = ss2.reshape(NPAD)

    x2p = _sc_l2(dst, src, edge_attr, sd1, ss1, s1, sd2, ss2, h2p)

    xf = jnp.transpose(x2p, (1, 0, 2)).reshape(1, NPAD * P * H2)
    out = _tc_head(xf, fc1_w, fc1_b.reshape(1, 20),
                   fc2_w, fc2_b.reshape(1, 16), fc3_w, fc3_b.reshape(1, 20),
                   fc4_w, fc4_b.reshape(1, 2))
    reg = jnp.zeros((1,), jnp.float32)
    return (out, reg)


# speedup vs baseline: 3.6979x; 1.0154x over previous
import functools

import jax
import jax.numpy as jnp
from jax import lax
from jax.experimental import pallas as pl
from jax.experimental.pallas import tpu as pltpu
from jax.experimental.pallas import tpu_sc as plsc

N = 10000
E = 320000
F_IN = 128
P = 16
H1 = 16
H2 = 4

NC = 2
NS = 16
L = 16
EPT = E // NS
NPAD = 10240
RPS = NPAD // NS
C1 = 80
NCH1 = EPT // C1
C2 = 400
NCH2 = EPT // C2
EPS = 1e-16

_params = pltpu.CompilerParams(needs_layout_passes=False,
                               use_tc_tiling_on_sc=False)


def _leaky(v):
    return jnp.where(v > 0, v, v * jnp.float32(0.2))


def _elu(v):
    return jnp.where(v > 0, v, jnp.exp(v) - jnp.float32(1.0))


def _sc_l1_body(dst_h, src_h, ea_h, sd_h, ss_h, h_h, x_out, s_out,
                x_sh, s_sh, sd_t, ss_t, dst_v, src_v, ea_v, score_v,
                hrow_v, ex_v, msg_v):
    cid = lax.axis_index("c")
    sid = lax.axis_index("s")
    coff = cid * (P // NC)
    z = jnp.zeros((L,), jnp.float32)

    def zb(i, _):
        ex_v[i, :] = z
        for j in range(8):
            msg_v[i, pl.ds(j * L, L)] = z
        return 0
    lax.fori_loop(0, C1, zb, 0)
    r0 = sid * RPS
    for t in range(RPS // C1):
        pltpu.sync_copy(msg_v, x_sh.at[pl.ds(r0 + t * C1, C1)])
        pltpu.sync_copy(ex_v, s_sh.at[pl.ds(r0 + t * C1, C1)])
    nn = sd_h.shape[0]
    pltpu.sync_copy(sd_h, sd_t.at[pl.ds(0, nn)])
    pltpu.sync_copy(ss_h, ss_t.at[pl.ds(0, nn)])
    plsc.subcore_barrier()

    base = sid * EPT

    def chunk(k, _):
        off = base + k * C1
        pltpu.sync_copy(dst_h.at[pl.ds(off, C1)], dst_v)
        pltpu.sync_copy(src_h.at[pl.ds(off, C1)], src_v)
        pltpu.sync_copy(ea_h.at[pl.ds(off, C1)], ea_v)
        pltpu.sync_copy(h_h.at[src_v], hrow_v)
        for g in range(C1 // L):
            d16 = dst_v[pl.ds(g * L, L)]
            s16 = src_v[pl.ds(g * L, L)]
            sc = plsc.load_gather(sd_t, [d16]) + plsc.load_gather(ss_t, [s16])
            score_v[pl.ds(g * L, L)] = _leaky(sc)

        def edge(e, _):
            e16 = jnp.full((L,), e, jnp.int32)
            sv = plsc.load_gather(score_v, [e16])
            ex_v[e, :] = jnp.exp(ea_v[e, :] * sv)
            hr = hrow_v[e, :]
            for j in range(8):
                j16 = jnp.full((L,), coff + j, jnp.int32)
                a = plsc.load_gather(ex_v, [e16, j16])
                msg_v[e, pl.ds(j * L, L)] = hr * a
            return 0
        lax.fori_loop(0, C1, edge, 0)
        pltpu.sync_copy(ex_v, s_sh.at[dst_v], add=True)
        pltpu.sync_copy(msg_v, x_sh.at[dst_v], add=True)
        return 0
    lax.fori_loop(0, NCH1, chunk, 0)
    plsc.subcore_barrier()

    for t in range(RPS // C1):
        rr = r0 + t * C1
        pltpu.sync_copy(x_sh.at[pl.ds(rr, C1)], msg_v)
        pltpu.sync_copy(s_sh.at[pl.ds(rr, C1)], ex_v)

        def nrm(r, _):
            r16 = jnp.full((L,), r, jnp.int32)
            for j in range(8):
                j16 = jnp.full((L,), coff + j, jnp.int32)
                sv = plsc.load_gather(ex_v, [r16, j16]) + jnp.float32(EPS)
                msg_v[r, pl.ds(j * L, L)] = msg_v[r, pl.ds(j * L, L)] / sv
            return 0
        lax.fori_loop(0, C1, nrm, 0)
        pltpu.sync_copy(msg_v, x_out.at[cid, pl.ds(rr, C1)])

    @pl.when(cid == 0)
    def _():
        pltpu.sync_copy(s_sh.at[pl.ds(r0, RPS)], s_out.at[pl.ds(r0, RPS)])


def _make_sc_l1(mesh):
    return pl.kernel(
        _sc_l1_body,
        out_type=(jax.ShapeDtypeStruct((NC, NPAD, (P // NC) * H1), jnp.float32),
                  jax.ShapeDtypeStruct((NPAD, P), jnp.float32)),
        mesh=mesh,
        compiler_params=_params,
        scratch_types=[
            pltpu.VMEM_SHARED((NPAD, (P // NC) * H1), jnp.float32),
            pltpu.VMEM_SHARED((NPAD, P), jnp.float32),
            pltpu.VMEM((NPAD,), jnp.float32),
            pltpu.VMEM((NPAD,), jnp.float32),
            pltpu.VMEM((C1,), jnp.int32),
            pltpu.VMEM((C1,), jnp.int32),
            pltpu.VMEM((C1, P), jnp.float32),
            pltpu.VMEM((C1,), jnp.float32),
            pltpu.VMEM((C1, H1), jnp.float32),
            pltpu.VMEM((C1, P), jnp.float32),
            pltpu.VMEM((C1, (P // NC) * H1), jnp.float32),
        ],
    )


def _sc_l2_body(dst_h, src_h, ea_h, sd1_h, ss1_h, s1_h, sd2_h, ss2_h, h_h,
                x_out, x_sh, s_sh, sd1_t, ss1_t, sd2_t, ss2_t,
                dst_v, src_v, ea_v, sc1_v, sc2_v, s1row_v, hrow_v,
                ex_v, msg_v):
    cid = lax.axis_index("c")
    sid = lax.axis_index("s")
    coff = cid * (P // NC)
    z = jnp.zeros((L,), jnp.float32)
    iota = lax.iota(jnp.int32, L)
    rep4 = iota // 4
    tile4 = iota % 4

    def zb(i, _):
        ex_v[i, :] = z
        msg_v[i, pl.ds(0, L)] = z
        msg_v[i, pl.ds(L, L)] = z
        return 0
    lax.fori_loop(0, C2, zb, 0)
    r0 = sid * RPS
    pltpu.sync_copy(msg_v, x_sh.at[pl.ds(r0, C2)])
    pltpu.sync_copy(msg_v.at[pl.ds(0, RPS - C2)],
                    x_sh.at[pl.ds(r0 + C2, RPS - C2)])
    pltpu.sync_copy(ex_v, s_sh.at[pl.ds(r0, C2)])
    pltpu.sync_copy(ex_v.at[pl.ds(0, RPS - C2)],
                    s_sh.at[pl.ds(r0 + C2, RPS - C2)])
    n1 = sd1_h.shape[0]
    pltpu.sync_copy(sd1_h, sd1_t.at[pl.ds(0, n1)])
    pltpu.sync_copy(ss1_h, ss1_t.at[pl.ds(0, n1)])
    pltpu.sync_copy(sd2_h, sd2_t)
    pltpu.sync_copy(ss2_h, ss2_t)
    plsc.subcore_barrier()

    base = sid * EPT

    def chunk(k, _):
        off = base + k * C2
        pltpu.sync_copy(dst_h.at[pl.ds(off, C2)], dst_v)
        pltpu.sync_copy(src_h.at[pl.ds(off, C2)], src_v)
        pltpu.sync_copy(ea_h.at[pl.ds(off, C2)], ea_v)
        pltpu.sync_copy(s1_h.at[dst_v], s1row_v)
        pltpu.sync_copy(h_h.at[src_v], hrow_v)
        for g in range(C2 // L):
            d16 = dst_v[pl.ds(g * L, L)]
            s16 = src_v[pl.ds(g * L, L)]
            c1 = plsc.load_gather(sd1_t, [d16]) + plsc.load_gather(ss1_t, [s16])
            sc1_v[pl.ds(g * L, L)] = _leaky(c1)
            c2 = plsc.load_gather(sd2_t, [d16]) + plsc.load_gather(ss2_t, [s16])
            sc2_v[pl.ds(g * L, L)] = _leaky(c2)

        def edge(e, _):
            e16 = jnp.full((L,), e, jnp.int32)
            sv1 = plsc.load_gather(sc1_v, [e16])
            ex1 = jnp.exp(ea_v[e, :] * sv1)
            al1 = ex1 / (s1row_v[e, :] + jnp.float32(EPS))
            sv2 = plsc.load_gather(sc2_v, [e16])
            ex_v[e, :] = jnp.exp(al1 * sv2)
            hv = plsc.load_gather(hrow_v, [e16, tile4])
            a0 = plsc.load_gather(ex_v, [e16, coff + rep4])
            a1 = plsc.load_gather(ex_v, [e16, coff + 4 + rep4])
            msg_v[e, pl.ds(0, L)] = a0 * hv
            msg_v[e, pl.ds(L, L)] = a1 * hv
            return 0
        lax.fori_loop(0, C2, edge, 0)
        pltpu.sync_copy(ex_v, s_sh.at[dst_v], add=True)
        pltpu.sync_copy(msg_v, x_sh.at[dst_v], add=True)
        return 0
    lax.fori_loop(0, NCH2, chunk, 0)
    plsc.subcore_barrier()

    sizes = (C2, RPS - C2)
    offs = (0, C2)
    for t in range(2):
        rr = r0 + offs[t]
        sz = sizes[t]
        pltpu.sync_copy(x_sh.at[pl.ds(rr, sz)], msg_v.at[pl.ds(0, sz)])
        pltpu.sync_copy(s_sh.at[pl.ds(rr, sz)], ex_v.at[pl.ds(0, sz)])

        def nrm(r, _):
            r16 = jnp.full((L,), r, jnp.int32)
            for j in range(2):
                jrep = coff + j * 4 + rep4
                sv = plsc.load_gather(ex_v, [r16, jrep]) + jnp.float32(EPS)
                msg_v[r, pl.ds(j * L, L)] = msg_v[r, pl.ds(j * L, L)] / sv
            return 0
        lax.fori_loop(0, sz, nrm, 0)
        pltpu.sync_copy(msg_v.at[pl.ds(0, sz)],
                        x_out.at[pl.ds(rr, sz), pl.ds(cid * 32, 32)])


def _make_sc_l2(mesh):
    return pl.kernel(
        _sc_l2_body,
        out_type=jax.ShapeDtypeStruct((NPAD, P * H2), jnp.float32),
        mesh=mesh,
        compiler_params=_params,
        scratch_types=[
            pltpu.VMEM_SHARED((NPAD, (P // NC) * H2), jnp.float32),
            pltpu.VMEM_SHARED((NPAD, P), jnp.float32),
            pltpu.VMEM((NPAD,), jnp.float32),
            pltpu.VMEM((NPAD,), jnp.float32),
            pltpu.VMEM((NPAD,), jnp.float32),
            pltpu.VMEM((NPAD,), jnp.float32),
            pltpu.VMEM((C2,), jnp.int32),
            pltpu.VMEM((C2,), jnp.int32),
            pltpu.VMEM((C2, P), jnp.float32),
            pltpu.VMEM((C2,), jnp.float32),
            pltpu.VMEM((C2,), jnp.float32),
            pltpu.VMEM((C2, P), jnp.float32),
            pltpu.VMEM((C2, P), jnp.float32),
            pltpu.VMEM((C2, P), jnp.float32),
            pltpu.VMEM((C2, (P // NC) * H2), jnp.float32),
        ],
    )


@functools.lru_cache(maxsize=None)
def _sc_kernels():
    mesh = plsc.VectorSubcoreMesh(core_axis_name="c", subcore_axis_name="s",
                                  num_cores=NC, num_subcores=NS)
    return (_make_sc_l1(mesh), _make_sc_l2(mesh))


NB = 10
NBK = N // NB
NBKP = NPAD // NB


def _tc1_body(x_ref, w_ref, adt_ref, h_ref, sd_ref, ss_ref):
    h = jnp.dot(x_ref[...], w_ref[...], preferred_element_type=jnp.float32)
    h_ref[...] = h
    sdss = lax.dot_general(h, adt_ref[...], (((1,), (1,)), ((), ())),
                           preferred_element_type=jnp.float32)
    sd_ref[...] = sdss[:, 0:1]
    ss_ref[...] = sdss[:, 1:2]


def _tc1(x, W1, adt):
    return pl.pallas_call(
        _tc1_body,
        grid=(NB,),
        in_specs=[
            pl.BlockSpec((NBK, F_IN), lambda i: (i, 0)),
            pl.BlockSpec((F_IN, H1), lambda i: (0, 0)),
            pl.BlockSpec((2, H1), lambda i: (0, 0)),
        ],
        out_specs=[
            pl.BlockSpec((NBK, H1), lambda i: (i, 0)),
            pl.BlockSpec((NBK, 1), lambda i: (i, 0)),
            pl.BlockSpec((NBK, 1), lambda i: (i, 0)),
        ],
        out_shape=[
            jax.ShapeDtypeStruct((N, H1), jnp.float32),
            jax.ShapeDtypeStruct((N, 1), jnp.float32),
            jax.ShapeDtypeStruct((N, 1), jnp.float32),
        ],
    )(x, W1, adt)


def _tc3_body(p_ref, w2_ref, adt2_ref, h2p_ref, sd_ref, ss_ref):
    a = _elu(p_ref[0])
    b = _elu(p_ref[1])
    h2 = (jnp.dot(a, w2_ref[0:128, :], preferred_element_type=jnp.float32)
          + jnp.dot(b, w2_ref[128:256, :], preferred_element_type=jnp.float32))
    h2p_ref[...] = jnp.concatenate(
        [h2, jnp.zeros((NBKP, P - H2), jnp.float32)], axis=1)
    sdss = lax.dot_general(h2, adt2_ref[...], (((1,), (1,)), ((), ())),
                           preferred_element_type=jnp.float32)
    sd_ref[...] = sdss[:, 0:1]
    ss_ref[...] = sdss[:, 1:2]


def _tc3(x1p, W2, adt2):
    return pl.pallas_call(
        _tc3_body,
        grid=(NB,),
        in_specs=[
            pl.BlockSpec((NC, NBKP, (P // NC) * H1), lambda i: (0, i, 0)),
            pl.BlockSpec((P * H1, H2), lambda i: (0, 0)),
            pl.BlockSpec((2, H2), lambda i: (0, 0)),
        ],
        out_specs=[
            pl.BlockSpec((NBKP, P), lambda i: (i, 0)),
            pl.BlockSpec((NBKP, 1), lambda i: (i, 0)),
            pl.BlockSpec((NBKP, 1), lambda i: (i, 0)),
        ],
        out_shape=[
            jax.ShapeDtypeStruct((NPAD, P), jnp.float32),
            jax.ShapeDtypeStruct((NPAD, 1), jnp.float32),
            jax.ShapeDtypeStruct((NPAD, 1), jnp.float32),
        ],
    )(x1p, W2, adt2)


NBH = 20
KB = (N * P * H2) // NBH


def _head_body(xf_ref, w1_ref, b1_ref, w2_ref, b2_ref, w3_ref, b3_ref,
               w4_ref, b4_ref, out_ref, acc_ref):
    i = pl.program_id(0)
    v = _elu(xf_ref[...])
    p = jnp.dot(v, w1_ref[...], preferred_element_type=jnp.float32)

    @pl.when(i == 0)
    def _():
        acc_ref[...] = p

    @pl.when(i > 0)
    def _():
        acc_ref[...] = acc_ref[...] + p

    @pl.when(i == NBH - 1)
    def _():
        zz = _elu(acc_ref[...] + b1_ref[...])
        zz = _elu(jnp.dot(zz, w2_ref[...], preferred_element_type=jnp.float32)
                  + b2_ref[...])
        zz = _elu(jnp.dot(zz, w3_ref[...], preferred_element_type=jnp.float32)
                  + b3_ref[...])
        out_ref[...] = (jnp.dot(zz, w4_ref[...],
                                preferred_element_type=jnp.float32)
                        + b4_ref[...])


def _tc_head(xf, w1, b1, w2, b2, w3, b3, w4, b4):
    return pl.pallas_call(
        _head_body,
        grid=(NBH,),
        in_specs=[
            pl.BlockSpec((1, KB), lambda i: (0, i)),
            pl.BlockSpec((KB, 20), lambda i: (i, 0)),
            pl.BlockSpec((1, 20), lambda i: (0, 0)),
            pl.BlockSpec((20, 16), lambda i: (0, 0)),
            pl.BlockSpec((1, 16), lambda i: (0, 0)),
            pl.BlockSpec((16, 20), lambda i: (0, 0)),
            pl.BlockSpec((1, 20), lambda i: (0, 0)),
            pl.BlockSpec((20, 2), lambda i: (0, 0)),
            pl.BlockSpec((1, 2), lambda i: (0, 0)),
        ],
        out_specs=pl.BlockSpec((1, 2), lambda i: (0, 0)),
        out_shape=jax.ShapeDtypeStruct((1, 2), jnp.float32),
        scratch_shapes=[pltpu.VMEM((1, 20), jnp.float32)],
    )(xf, w1, b1, w2, b2, w3, b3, w4, b4)


def kernel(x, edge_index, edge_attr, y, W1, att1, W2, att2,
           fc1_w, fc1_b, fc2_w, fc2_b, fc3_w, fc3_b, fc4_w, fc4_b):
    src = edge_index[0]
    dst = edge_index[1]
    _sc_l1, _sc_l2 = _sc_kernels()

    h1, sd1, ss1 = _tc1(x, W1, att1.reshape(2, H1))
    sd1 = sd1.reshape(N)
    ss1 = ss1.reshape(N)

    x1p, s1 = _sc_l1(dst, src, edge_attr, sd1, ss1, h1)

    h2p, sd2, ss2 = _tc3(x1p, W2, att2.reshape(2, H2))
    sd2 = sd2.reshape(NPAD)
    ss2 = ss2.reshape(NPAD)

    x2 = _sc_l2(dst, src, edge_attr, sd1, ss1, s1, sd2, ss2, h2p)
    xf = x2.reshape(1, NPAD * P * H2)
    out = _tc_head(xf, fc1_w, fc1_b.reshape(1, 20),
                   fc2_w, fc2_b.reshape(1, 16), fc3_w, fc3_b.reshape(1, 20),
                   fc4_w, fc4_b.reshape(1, 2))
    reg = jnp.zeros((1,), jnp.float32)
    return (out, reg)
